# Initial kernel scaffold; baseline (speedup 1.0000x reference)
import jax
import jax.numpy as jnp
from jax.experimental import pallas as pl


def kernel(x, emb):
    raise NotImplementedError("write your pallas kernel here")


# speedup vs baseline: 1.5885x; 1.5885x over previous
import functools

import jax
import jax.numpy as jnp
from jax import lax
from jax.experimental import pallas as pl
from jax.experimental.pallas import tpu as pltpu
from jax.experimental.pallas import tpu_sc as plsc

_NC = 2
_NS = 16
_NW = _NC * _NS

_CHUNK = 32


def _copy_body(n_chunks, emb, out, buf0, buf1, sem0, sem1):
    wid = lax.axis_index("s") * _NC + lax.axis_index("c")
    base = wid * (n_chunks * _CHUNK)
    bufs = (buf0, buf1)
    sems = (sem0, sem1)
    cps = [None, None]
    cps[0] = pltpu.async_copy(emb.at[pl.ds(base, _CHUNK)], buf0, sem0)
    for c in range(n_chunks):
        if c + 1 < n_chunks:
            j = (c + 1) % 2
            cps[j] = pltpu.async_copy(
                emb.at[pl.ds(base + (c + 1) * _CHUNK, _CHUNK)], bufs[j], sems[j]
            )
        i = c % 2
        cps[i].wait()
        pltpu.sync_copy(bufs[i], out.at[pl.ds(base + c * _CHUNK, _CHUNK)])


def kernel(x, emb):
    seq = x.shape[1]
    dim = emb.shape[1]
    n_chunks = seq // (_NW * _CHUNK)
    mesh = plsc.VectorSubcoreMesh(core_axis_name="c", subcore_axis_name="s")
    run = pl.kernel(
        functools.partial(_copy_body, n_chunks),
        out_type=jax.ShapeDtypeStruct((seq, dim), emb.dtype),
        mesh=mesh,
        scratch_types=[
            pltpu.VMEM((_CHUNK, dim), emb.dtype),
            pltpu.VMEM((_CHUNK, dim), emb.dtype),
            pltpu.SemaphoreType.DMA,
            pltpu.SemaphoreType.DMA,
        ],
    )
    return run(emb)
